# Initial kernel scaffold; baseline (speedup 1.0000x reference)
import functools

import jax
import jax.numpy as jnp
from jax import lax
from jax.experimental import pallas as pl
from jax.experimental.pallas import tpu as pltpu
from jax.experimental.pallas import tpu_sc as plsc

N = 100000
K = 128
GROUP = 32
NG = K // GROUP
B = 16384

NC = 2
NS = 16
NW = NC * NS
BPW = B // NW
CHUNK = 128
NCHUNK = BPW // CHUNK
LANES = 16


def _sc_kernel(x_hbm, q_hbm, s_hbm, out_hbm, idx_v, q_v, s_v, o_v, sem):
    wid = lax.axis_index("s") * NC + lax.axis_index("c")
    base = wid * BPW

    for c in range(NCHUNK):
        pltpu.sync_copy(x_hbm.at[pl.ds(base + c * CHUNK, CHUNK)], idx_v.at[c])

    def do_chunk(c, carry):
        cq = pltpu.async_copy(q_hbm.at[idx_v.at[c]], q_v, sem)
        cs = pltpu.async_copy(s_hbm.at[idx_v.at[c]], s_v, sem)
        cq.wait()
        cs.wait()

        def row_body(r, rc):
            for g in range(NG):
                scale = s_v[r, g]
                for h in range(2):
                    j = g * 2 + h
                    q16 = q_v[r, pl.ds(j * LANES, LANES)]
                    o_v[r, pl.ds(j * LANES, LANES)] = (
                        q16.astype(jnp.float32) * scale)
            return rc

        lax.fori_loop(0, CHUNK, row_body, 0)
        pltpu.sync_copy(o_v, out_hbm.at[pl.ds(base + c * CHUNK, CHUNK)])
        return carry

    lax.fori_loop(0, NCHUNK, do_chunk, 0, unroll=True)


def kernel(x, qweights, scales):
    mesh = plsc.VectorSubcoreMesh(core_axis_name="c", subcore_axis_name="s")
    run = functools.partial(
        pl.kernel,
        mesh=mesh,
        out_type=jax.ShapeDtypeStruct((B, K), jnp.float32),
        scratch_types=[
            pltpu.VMEM((NCHUNK, CHUNK), jnp.int32),
            pltpu.VMEM((CHUNK, K), jnp.int32),
            pltpu.VMEM((CHUNK, NG), jnp.float32),
            pltpu.VMEM((CHUNK, K), jnp.float32),
            pltpu.SemaphoreType.DMA,
        ],
    )(_sc_kernel)
    return run(x, qweights, scales)


# speedup vs baseline: 1.0438x; 1.0438x over previous
import functools

import jax
import jax.numpy as jnp
from jax import lax
from jax.experimental import pallas as pl
from jax.experimental.pallas import tpu as pltpu
from jax.experimental.pallas import tpu_sc as plsc

N = 100000
K = 128
GROUP = 32
NG = K // GROUP
B = 16384

NC = 2
NS = 16
NW = NC * NS
BPW = B // NW
CHUNK = 128
NCHUNK = BPW // CHUNK
LANES = 16


def _sc_kernel(x_hbm, q_hbm, s_hbm, out_hbm, idx_v, q_v, sidx_v, s4_v, o_v,
               sem):
    wid = lax.axis_index("s") * NC + lax.axis_index("c")
    base = wid * BPW

    for c in range(NCHUNK):
        pltpu.sync_copy(x_hbm.at[pl.ds(base + c * CHUNK, CHUNK)], idx_v.at[c])

    def do_chunk(c, carry):
        cq = pltpu.async_copy(q_hbm.at[idx_v.at[c]], q_v, sem)

        for t in range(CHUNK // LANES):
            tok = idx_v[c, pl.ds(t * LANES, LANES)]
            tok4 = tok * NG
            for g in range(NG):
                sidx_v[g, pl.ds(t * LANES, LANES)] = tok4 + g

        copies = [
            pltpu.async_copy(s_hbm.at[sidx_v.at[g]],
                             s4_v.at[pl.ds(g * CHUNK, CHUNK)], sem)
            for g in range(NG)
        ]
        cq.wait()
        for cp in copies:
            cp.wait()

        def block_body(t, rc):
            rbase = t * LANES
            for g in range(NG):
                s16 = s4_v[pl.ds(g * CHUNK + rbase, LANES)]
                for l in range(LANES):
                    lidx = jnp.full((LANES,), l, jnp.int32)
                    svec = jnp.take_along_axis(
                        s16, lidx, axis=0, mode="promise_in_bounds")
                    r = rbase + l
                    for h in range(2):
                        j = g * 2 + h
                        q16 = q_v[r, pl.ds(j * LANES, LANES)]
                        o_v[r, pl.ds(j * LANES, LANES)] = (
                            q16.astype(jnp.float32) * svec)
            return rc

        lax.fori_loop(0, CHUNK // LANES, block_body, 0)
        pltpu.sync_copy(o_v, out_hbm.at[pl.ds(base + c * CHUNK, CHUNK)])
        return carry

    lax.fori_loop(0, NCHUNK, do_chunk, 0, unroll=True)


def kernel(x, qweights, scales):
    mesh = plsc.VectorSubcoreMesh(core_axis_name="c", subcore_axis_name="s")
    run = functools.partial(
        pl.kernel,
        mesh=mesh,
        out_type=jax.ShapeDtypeStruct((B, K), jnp.float32),
        scratch_types=[
            pltpu.VMEM((NCHUNK, CHUNK), jnp.int32),
            pltpu.VMEM((CHUNK, K), jnp.int32),
            pltpu.VMEM((NG, CHUNK), jnp.int32),
            pltpu.VMEM((NG * CHUNK,), jnp.float32),
            pltpu.VMEM((CHUNK, K), jnp.float32),
            pltpu.SemaphoreType.DMA,
        ],
    )(_sc_kernel)
    return run(x, qweights, scales.reshape(N * NG))


# speedup vs baseline: 2.8280x; 2.7093x over previous
import functools

import jax
import jax.numpy as jnp
from jax import lax
from jax.experimental import pallas as pl
from jax.experimental.pallas import tpu as pltpu
from jax.experimental.pallas import tpu_sc as plsc

N = 100000
K = 128
GROUP = 32
NG = K // GROUP
B = 16384

NC = 2
NS = 16
NW = NC * NS
BPW = B // NW
CHUNK = 128
NCHUNK = BPW // CHUNK
LANES = 16


def _sc_kernel(x_hbm, q_hbm, s_hbm, out_hbm, idx_v, q_v, sidx_v, s4_v, o_v, sem):
    wid = lax.axis_index("s") * NC + lax.axis_index("c")
    base = wid * BPW

    for c in range(NCHUNK):
        pltpu.sync_copy(x_hbm.at[pl.ds(base + c * CHUNK, CHUNK)], idx_v.at[c])

    def do_chunk(c, carry):
        cq = pltpu.async_copy(q_hbm.at[idx_v.at[c]], q_v, sem)

        for t in range(CHUNK // LANES):
            tok = idx_v[c, pl.ds(t * LANES, LANES)]
            for g in range(NG):
                sidx_v[g, pl.ds(t * LANES, LANES)] = tok + (g * N)

        copies = [
            pltpu.async_copy(s_hbm.at[sidx_v.at[g]],
                             s4_v.at[pl.ds(g * CHUNK, CHUNK)], sem)
            for g in range(NG)
        ]
        cq.wait()
        for cp in copies:
            cp.wait()

        def blk_body(t, rc):
            rbase = t * LANES
            for g in range(NG):
                s16 = s4_v[pl.ds(g * CHUNK + rbase, LANES)]
                for l in range(LANES):
                    lidx = jnp.full((LANES,), l, jnp.int32)
                    svec = jnp.take_along_axis(
                        s16, lidx, axis=0, mode="promise_in_bounds")
                    r = rbase + l
                    for h in range(2):
                        j = g * 2 + h
                        q16 = q_v[r, pl.ds(j * LANES, LANES)]
                        o_v[r, pl.ds(j * LANES, LANES)] = (
                            q16.astype(jnp.float32) * svec)
            return rc

        lax.fori_loop(0, CHUNK // LANES, blk_body, 0)
        pltpu.sync_copy(o_v, out_hbm.at[pl.ds(base + c * CHUNK, CHUNK)])
        return carry

    lax.fori_loop(0, NCHUNK, do_chunk, 0, unroll=True)


def kernel(x, qweights, scales):
    mesh = plsc.VectorSubcoreMesh(core_axis_name="c", subcore_axis_name="s")
    run = functools.partial(
        pl.kernel,
        mesh=mesh,
        out_type=jax.ShapeDtypeStruct((B, K), jnp.float32),
        scratch_types=[
            pltpu.VMEM((NCHUNK, CHUNK), jnp.int32),
            pltpu.VMEM((CHUNK, K), jnp.int32),
            pltpu.VMEM((NG, CHUNK), jnp.int32),
            pltpu.VMEM((NG * CHUNK,), jnp.float32),
            pltpu.VMEM((CHUNK, K), jnp.float32),
            pltpu.SemaphoreType.DMA,
        ],
    )(_sc_kernel)
    return run(x, qweights, scales.T.reshape(N * NG))
